# baseline (device time: 734834 ns/iter reference)
import jax
import jax.numpy as jnp
from jax import lax
from jax.experimental import pallas as pl
from jax.experimental.pallas import tpu as pltpu

N_DEV = 4


def _all_gather(x):
    m_per, k = x.shape

    def body(x_hbm, xg_hbm, copy_sem, send_sems, recv_sems):
        my = lax.axis_index("i")
        right = lax.rem(my + 1, N_DEV)

        cp = pltpu.make_async_copy(x_hbm, xg_hbm.at[my], copy_sem)
        cp.start()
        cp.wait()

        for h in range(N_DEV - 1):
            slot = lax.rem(my - h + N_DEV, N_DEV)
            rdma = pltpu.make_async_remote_copy(
                src_ref=xg_hbm.at[slot],
                dst_ref=xg_hbm.at[slot],
                send_sem=send_sems.at[h],
                recv_sem=recv_sems.at[h],
                device_id=(right,),
                device_id_type=pl.DeviceIdType.MESH,
            )
            rdma.start()
            rdma.wait()

    return pl.pallas_call(
        body,
        out_shape=jax.ShapeDtypeStruct((N_DEV, m_per, k), x.dtype),
        in_specs=[pl.BlockSpec(memory_space=pltpu.MemorySpace.HBM)],
        out_specs=pl.BlockSpec(memory_space=pltpu.MemorySpace.HBM),
        scratch_shapes=[
            pltpu.SemaphoreType.DMA,
            pltpu.SemaphoreType.DMA((N_DEV - 1,)),
            pltpu.SemaphoreType.DMA((N_DEV - 1,)),
        ],
    )(x)


_BM = 256
_BN = 512


def _gemm(xg, w):
    m, k = xg.shape
    _, n = w.shape
    nr = m // _BM
    nc = n // _BN

    def body(x_ref, w_ref, y_ref, amax_ref, acc_ref):
        c = pl.program_id(0)
        r = pl.program_id(1)
        y = jnp.dot(x_ref[...], w_ref[...], preferred_element_type=jnp.float32)
        y_ref[...] = y
        m_blk = jnp.max(jnp.abs(y))

        @pl.when(jnp.logical_and(c == 0, r == 0))
        def _():
            acc_ref[0] = m_blk

        @pl.when(jnp.logical_not(jnp.logical_and(c == 0, r == 0)))
        def _():
            acc_ref[0] = jnp.maximum(acc_ref[0], m_blk)

        @pl.when(jnp.logical_and(c == nc - 1, r == nr - 1))
        def _():
            amax_ref[0, 0] = acc_ref[0]

    y, amax = pl.pallas_call(
        body,
        grid=(nc, nr),
        in_specs=[
            pl.BlockSpec((_BM, k), lambda c, r: (r, 0)),
            pl.BlockSpec((k, _BN), lambda c, r: (0, c)),
        ],
        out_specs=[
            pl.BlockSpec((_BM, _BN), lambda c, r: (r, c)),
            pl.BlockSpec((1, 1), lambda c, r: (0, 0), memory_space=pltpu.MemorySpace.SMEM),
        ],
        out_shape=[
            jax.ShapeDtypeStruct((m, n), jnp.float32),
            jax.ShapeDtypeStruct((1, 1), jnp.float32),
        ],
        scratch_shapes=[pltpu.SMEM((1,), jnp.float32)],
    )(xg, w)
    return y, amax


def _amax_allreduce(amax_local):
    def body(a_ref, out_ref, vbuf, gbuf, copy_sem, send_sems, recv_sems):
        my = lax.axis_index("i")
        s = a_ref[0, 0]
        vbuf[...] = jnp.full((8, 128), s, jnp.float32)

        cp = pltpu.make_async_copy(vbuf, gbuf.at[my], copy_sem)
        cp.start()
        cp.wait()

        sends = []
        for d in range(1, N_DEV):
            tgt = lax.rem(my + d, N_DEV)
            rdma = pltpu.make_async_remote_copy(
                src_ref=vbuf,
                dst_ref=gbuf.at[my],
                send_sem=send_sems.at[d - 1],
                recv_sem=recv_sems.at[d - 1],
                device_id=(tgt,),
                device_id_type=pl.DeviceIdType.MESH,
            )
            rdma.start()
            sends.append(rdma)
        for rdma in sends:
            rdma.wait_send()
        for d in range(1, N_DEV):
            src_slot = lax.rem(my - d + N_DEV, N_DEV)
            rcv = pltpu.make_async_remote_copy(
                src_ref=vbuf,
                dst_ref=gbuf.at[src_slot],
                send_sem=send_sems.at[d - 1],
                recv_sem=recv_sems.at[d - 1],
                device_id=(my,),
                device_id_type=pl.DeviceIdType.MESH,
            )
            rcv.wait_recv()
        out_ref[0, 0] = jnp.max(gbuf[...])

    return pl.pallas_call(
        body,
        out_shape=jax.ShapeDtypeStruct((1, 1), jnp.float32),
        in_specs=[pl.BlockSpec(memory_space=pltpu.MemorySpace.SMEM)],
        out_specs=pl.BlockSpec(memory_space=pltpu.MemorySpace.SMEM),
        scratch_shapes=[
            pltpu.VMEM((8, 128), jnp.float32),
            pltpu.VMEM((N_DEV, 8, 128), jnp.float32),
            pltpu.SemaphoreType.DMA,
            pltpu.SemaphoreType.DMA((N_DEV - 1,)),
            pltpu.SemaphoreType.DMA((N_DEV - 1,)),
        ],
    )(amax_local)


_BQ = 512


def _quantize(y, amax_g):
    m, n = y.shape
    nb = m // _BQ

    def body(a_ref, y_ref, o_ref):
        s = a_ref[0, 0] / 448.0
        z = jnp.clip(y_ref[...] / s, -448.0, 448.0)
        q = z.astype(jnp.float8_e4m3fn).astype(jnp.float32)
        o_ref[...] = q * s

    return pl.pallas_call(
        body,
        grid=(nb,),
        in_specs=[
            pl.BlockSpec((1, 1), lambda b: (0, 0), memory_space=pltpu.MemorySpace.SMEM),
            pl.BlockSpec((_BQ, n), lambda b: (b, 0)),
        ],
        out_specs=pl.BlockSpec((_BQ, n), lambda b: (b, 0)),
        out_shape=jax.ShapeDtypeStruct((m, n), jnp.float32),
    )(amax_g, y)


def kernel(x, w_mat):
    m_per, k = x.shape
    xg = _all_gather(x)
    xg = xg.reshape(N_DEV * m_per, k)
    y, amax_local = _gemm(xg, w_mat)
    amax_g = _amax_allreduce(amax_local)
    return _quantize(y, amax_g)


# device time: 456668 ns/iter; 1.6091x vs baseline; 1.6091x over previous
import jax
import jax.numpy as jnp
from jax import lax
from jax.experimental import pallas as pl
from jax.experimental.pallas import tpu as pltpu

N_DEV = 4


def _all_gather(x):
    m_per, k = x.shape
    half = m_per // 2

    def body(x_hbm, xg_hbm, copy_sems, send_sems, recv_sems):
        my = lax.axis_index("i")
        left = lax.rem(my - 1 + N_DEV, N_DEV)
        right = lax.rem(my + 1, N_DEV)
        opp = lax.rem(my + 2, N_DEV)

        def rc(src, dst, s, r, dev):
            return pltpu.make_async_remote_copy(
                src_ref=src, dst_ref=dst,
                send_sem=send_sems.at[s], recv_sem=recv_sems.at[r],
                device_id=(dev,), device_id_type=pl.DeviceIdType.MESH,
            )

        cps = []
        for j in range(2):
            cp = pltpu.make_async_copy(
                x_hbm.at[pl.ds(j * half, half)], xg_hbm.at[2 * my + j],
                copy_sems.at[j],
            )
            cp.start()
            cps.append(cp)

        sends = []
        for j in range(2):
            s = rc(x_hbm.at[pl.ds(j * half, half)], xg_hbm.at[2 * my + j],
                   j, j, right)
            s.start()
            sends.append(s)
        for j in range(2):
            s = rc(x_hbm.at[pl.ds(j * half, half)], xg_hbm.at[2 * my + j],
                   2 + j, 2 + j, left)
            s.start()
            sends.append(s)

        def recv(dst_slot, r):
            d = rc(xg_hbm.at[dst_slot], xg_hbm.at[dst_slot], 0, r, my)
            d.wait_recv()

        recv(2 * left, 0)
        f_r = rc(xg_hbm.at[2 * left], xg_hbm.at[2 * left], 4, 4, right)
        f_r.start()
        sends.append(f_r)

        recv(2 * right + 1, 3)
        f_l = rc(xg_hbm.at[2 * right + 1], xg_hbm.at[2 * right + 1], 5, 5, left)
        f_l.start()
        sends.append(f_l)

        recv(2 * left + 1, 1)
        recv(2 * right, 2)
        recv(2 * opp, 4)
        recv(2 * opp + 1, 5)

        for cp in cps:
            cp.wait()
        for s in sends:
            s.wait_send()

    return pl.pallas_call(
        body,
        out_shape=jax.ShapeDtypeStruct((2 * N_DEV, half, k), x.dtype),
        in_specs=[pl.BlockSpec(memory_space=pltpu.MemorySpace.HBM)],
        out_specs=pl.BlockSpec(memory_space=pltpu.MemorySpace.HBM),
        scratch_shapes=[
            pltpu.SemaphoreType.DMA((2,)),
            pltpu.SemaphoreType.DMA((6,)),
            pltpu.SemaphoreType.DMA((6,)),
        ],
    )(x)


_BM = 256
_BN = 512


def _gemm(xg, w):
    m, k = xg.shape
    _, n = w.shape
    nr = m // _BM
    nc = n // _BN

    def body(x_ref, w_ref, y_ref, amax_ref, acc_ref):
        c = pl.program_id(0)
        r = pl.program_id(1)
        y = jnp.dot(x_ref[...], w_ref[...], preferred_element_type=jnp.float32)
        y_ref[...] = y
        m_blk = jnp.max(jnp.abs(y))

        @pl.when(jnp.logical_and(c == 0, r == 0))
        def _():
            acc_ref[0] = m_blk

        @pl.when(jnp.logical_not(jnp.logical_and(c == 0, r == 0)))
        def _():
            acc_ref[0] = jnp.maximum(acc_ref[0], m_blk)

        @pl.when(jnp.logical_and(c == nc - 1, r == nr - 1))
        def _():
            amax_ref[0, 0] = acc_ref[0]

    y, amax = pl.pallas_call(
        body,
        grid=(nc, nr),
        in_specs=[
            pl.BlockSpec((_BM, k), lambda c, r: (r, 0)),
            pl.BlockSpec((k, _BN), lambda c, r: (0, c)),
        ],
        out_specs=[
            pl.BlockSpec((_BM, _BN), lambda c, r: (r, c)),
            pl.BlockSpec((1, 1), lambda c, r: (0, 0), memory_space=pltpu.MemorySpace.SMEM),
        ],
        out_shape=[
            jax.ShapeDtypeStruct((m, n), jnp.float32),
            jax.ShapeDtypeStruct((1, 1), jnp.float32),
        ],
        scratch_shapes=[pltpu.SMEM((1,), jnp.float32)],
    )(xg, w)
    return y, amax


def _amax_allreduce(amax_local):
    def body(a_ref, out_ref, vbuf, gbuf, copy_sem, send_sems, recv_sems):
        my = lax.axis_index("i")
        s = a_ref[0, 0]
        vbuf[...] = jnp.full((8, 128), s, jnp.float32)

        cp = pltpu.make_async_copy(vbuf, gbuf.at[my], copy_sem)
        cp.start()
        cp.wait()

        sends = []
        for d in range(1, N_DEV):
            tgt = lax.rem(my + d, N_DEV)
            rdma = pltpu.make_async_remote_copy(
                src_ref=vbuf,
                dst_ref=gbuf.at[my],
                send_sem=send_sems.at[d - 1],
                recv_sem=recv_sems.at[d - 1],
                device_id=(tgt,),
                device_id_type=pl.DeviceIdType.MESH,
            )
            rdma.start()
            sends.append(rdma)
        for rdma in sends:
            rdma.wait_send()
        for d in range(1, N_DEV):
            src_slot = lax.rem(my - d + N_DEV, N_DEV)
            rcv = pltpu.make_async_remote_copy(
                src_ref=vbuf,
                dst_ref=gbuf.at[src_slot],
                send_sem=send_sems.at[d - 1],
                recv_sem=recv_sems.at[d - 1],
                device_id=(my,),
                device_id_type=pl.DeviceIdType.MESH,
            )
            rcv.wait_recv()
        out_ref[0, 0] = jnp.max(gbuf[...])

    return pl.pallas_call(
        body,
        out_shape=jax.ShapeDtypeStruct((1, 1), jnp.float32),
        in_specs=[pl.BlockSpec(memory_space=pltpu.MemorySpace.SMEM)],
        out_specs=pl.BlockSpec(memory_space=pltpu.MemorySpace.SMEM),
        scratch_shapes=[
            pltpu.VMEM((8, 128), jnp.float32),
            pltpu.VMEM((N_DEV, 8, 128), jnp.float32),
            pltpu.SemaphoreType.DMA,
            pltpu.SemaphoreType.DMA((N_DEV - 1,)),
            pltpu.SemaphoreType.DMA((N_DEV - 1,)),
        ],
    )(amax_local)


_BQ = 512


def _quantize(y, amax_g):
    m, n = y.shape
    nb = m // _BQ

    def body(a_ref, y_ref, o_ref):
        s = a_ref[0, 0] / 448.0
        z = jnp.clip(y_ref[...] / s, -448.0, 448.0)
        q = z.astype(jnp.float8_e4m3fn).astype(jnp.float32)
        o_ref[...] = q * s

    return pl.pallas_call(
        body,
        grid=(nb,),
        in_specs=[
            pl.BlockSpec((1, 1), lambda b: (0, 0), memory_space=pltpu.MemorySpace.SMEM),
            pl.BlockSpec((_BQ, n), lambda b: (b, 0)),
        ],
        out_specs=pl.BlockSpec((_BQ, n), lambda b: (b, 0)),
        out_shape=jax.ShapeDtypeStruct((m, n), jnp.float32),
    )(amax_g, y)


def kernel(x, w_mat):
    m_per, k = x.shape
    xg = _all_gather(x)
    xg = xg.reshape(N_DEV * m_per, k)
    y, amax_local = _gemm(xg, w_mat)
    amax_g = _amax_allreduce(amax_local)
    return _quantize(y, amax_g)


# device time: 367601 ns/iter; 1.9990x vs baseline; 1.2423x over previous
import jax
import jax.numpy as jnp
from jax import lax
from jax.experimental import pallas as pl
from jax.experimental.pallas import tpu as pltpu

N_DEV = 4
HALF = 512
BN = 512


def kernel(x, w_mat):
    m_per, k = x.shape
    _, n_per = w_mat.shape
    m = N_DEV * m_per
    nc = n_per // BN

    def body(x_hbm, w_hbm, out_hbm, xg_hbm,
             wbuf, xbuf, ybuf, vbuf, gbuf, amax_s,
             ag_ss, ag_rs, xl_sem, wl_sem, yo_sems,
             acp_sem, am_ss, am_rs, ep_sems):
        my = lax.axis_index("i")
        left = lax.rem(my - 1 + N_DEV, N_DEV)
        right = lax.rem(my + 1, N_DEV)
        opp = lax.rem(my + 2, N_DEV)

        def rc(src, dst, s, r, dev):
            return pltpu.make_async_remote_copy(
                src_ref=src, dst_ref=dst,
                send_sem=ag_ss.at[s], recv_sem=ag_rs.at[r],
                device_id=(dev,), device_id_type=pl.DeviceIdType.MESH,
            )

        wcp = pltpu.make_async_copy(w_hbm, wbuf, wl_sem)
        wcp.start()

        sends = []
        for j in range(2):
            s = rc(x_hbm.at[pl.ds(j * HALF, HALF)], xg_hbm.at[2 * my + j],
                   j, j, right)
            s.start()
            sends.append(s)
        for j in range(2):
            s = rc(x_hbm.at[pl.ds(j * HALF, HALF)], xg_hbm.at[2 * my + j],
                   2 + j, 2 + j, left)
            s.start()
            sends.append(s)

        def wait_recv(slot, r):
            d = rc(xg_hbm.at[slot], xg_hbm.at[slot], 0, r, my)
            d.wait_recv()

        amax_s[0] = 0.0
        wcp.wait()

        phases = [
            (2 * my, None, x_hbm.at[pl.ds(0, HALF)]),
            (2 * my + 1, None, x_hbm.at[pl.ds(HALF, HALF)]),
            (2 * left, 0, None),
            (2 * right, 2, None),
            (2 * right + 1, 3, None),
            (2 * left + 1, 1, None),
            (2 * opp, 4, None),
            (2 * opp + 1, 5, None),
        ]

        youts = []
        for p, (slot, rs, src) in enumerate(phases):
            if rs is not None:
                wait_recv(slot, rs)
                src = xg_hbm.at[slot]
            if p == 2:
                f = rc(xg_hbm.at[2 * left], xg_hbm.at[2 * left], 4, 4, right)
                f.start()
                sends.append(f)
            if p == 4:
                f = rc(xg_hbm.at[2 * right + 1], xg_hbm.at[2 * right + 1],
                       5, 5, left)
                f.start()
                sends.append(f)

            xcp = pltpu.make_async_copy(src, xbuf, xl_sem)
            xcp.start()
            xcp.wait()

            if p >= 2:
                youts[p - 2].wait()
            b = p % 2
            for c in range(nc):
                ybuf[b, :, c * BN:(c + 1) * BN] = jnp.dot(
                    xbuf[...], wbuf[:, c * BN:(c + 1) * BN],
                    preferred_element_type=jnp.float32,
                )
            amax_s[0] = jnp.maximum(amax_s[0], jnp.max(jnp.abs(ybuf[b])))
            yo = pltpu.make_async_copy(
                ybuf.at[b], out_hbm.at[pl.ds(slot * HALF, HALF)], yo_sems.at[b]
            )
            yo.start()
            youts.append(yo)

        youts[-2].wait()
        youts[-1].wait()

        vbuf[...] = jnp.full((8, 128), amax_s[0], jnp.float32)
        acp = pltpu.make_async_copy(vbuf, gbuf.at[my], acp_sem)
        acp.start()
        acp.wait()
        asends = []
        for d in range(1, N_DEV):
            tgt = lax.rem(my + d, N_DEV)
            a = pltpu.make_async_remote_copy(
                src_ref=vbuf, dst_ref=gbuf.at[my],
                send_sem=am_ss.at[d - 1], recv_sem=am_rs.at[d - 1],
                device_id=(tgt,), device_id_type=pl.DeviceIdType.MESH,
            )
            a.start()
            asends.append(a)
        for a in asends:
            a.wait_send()
        for d in range(1, N_DEV):
            src_slot = lax.rem(my - d + N_DEV, N_DEV)
            a = pltpu.make_async_remote_copy(
                src_ref=vbuf, dst_ref=gbuf.at[src_slot],
                send_sem=am_ss.at[d - 1], recv_sem=am_rs.at[d - 1],
                device_id=(my,), device_id_type=pl.DeviceIdType.MESH,
            )
            a.wait_recv()
        scale = jnp.max(gbuf[...]) / 448.0

        stores = []
        for bq in range(2 * N_DEV):
            b = bq % 2
            if bq >= 2:
                stores[bq - 2].wait()
            ld = pltpu.make_async_copy(
                out_hbm.at[pl.ds(bq * HALF, HALF)], ybuf.at[b], ep_sems.at[b]
            )
            ld.start()
            ld.wait()
            z = jnp.clip(ybuf[b] / scale, -448.0, 448.0)
            ybuf[b] = z.astype(jnp.float8_e4m3fn).astype(jnp.float32) * scale
            st = pltpu.make_async_copy(
                ybuf.at[b], out_hbm.at[pl.ds(bq * HALF, HALF)], ep_sems.at[2 + b]
            )
            st.start()
            stores.append(st)
        stores[-2].wait()
        stores[-1].wait()

        for s in sends:
            s.wait_send()

    out, _ = pl.pallas_call(
        body,
        out_shape=[
            jax.ShapeDtypeStruct((m, n_per), jnp.float32),
            jax.ShapeDtypeStruct((2 * N_DEV, HALF, k), jnp.float32),
        ],
        in_specs=[
            pl.BlockSpec(memory_space=pltpu.MemorySpace.HBM),
            pl.BlockSpec(memory_space=pltpu.MemorySpace.HBM),
        ],
        out_specs=[
            pl.BlockSpec(memory_space=pltpu.MemorySpace.HBM),
            pl.BlockSpec(memory_space=pltpu.MemorySpace.HBM),
        ],
        scratch_shapes=[
            pltpu.VMEM((k, n_per), jnp.float32),
            pltpu.VMEM((HALF, k), jnp.float32),
            pltpu.VMEM((2, HALF, n_per), jnp.float32),
            pltpu.VMEM((8, 128), jnp.float32),
            pltpu.VMEM((N_DEV, 8, 128), jnp.float32),
            pltpu.SMEM((1,), jnp.float32),
            pltpu.SemaphoreType.DMA((6,)),
            pltpu.SemaphoreType.DMA((6,)),
            pltpu.SemaphoreType.DMA,
            pltpu.SemaphoreType.DMA,
            pltpu.SemaphoreType.DMA((2,)),
            pltpu.SemaphoreType.DMA,
            pltpu.SemaphoreType.DMA((3,)),
            pltpu.SemaphoreType.DMA((3,)),
            pltpu.SemaphoreType.DMA((4,)),
        ],
        compiler_params=pltpu.CompilerParams(
            vmem_limit_bytes=60 * 1024 * 1024,
        ),
    )(x, w_mat)
    return out


# device time: 343276 ns/iter; 2.1407x vs baseline; 1.0709x over previous
import jax
import jax.numpy as jnp
from jax import lax
from jax.experimental import pallas as pl
from jax.experimental.pallas import tpu as pltpu

N_DEV = 4
HALF = 512
QTR = 256


def kernel(x, w_mat):
    m_per, k = x.shape
    _, n_per = w_mat.shape
    m = N_DEV * m_per

    def body(x_hbm, w_hbm, out_hbm, xg_hbm,
             wbuf, xbuf, ybuf, vbuf, gbuf, amax_s,
             ag_ss, ag_rs, xl_sem, wl_sem, yo_sems,
             acp_sem, am_ss, am_rs, ep_ld, ep_st):
        my = lax.axis_index("i")
        left = lax.rem(my - 1 + N_DEV, N_DEV)
        right = lax.rem(my + 1, N_DEV)
        opp = lax.rem(my + 2, N_DEV)
        r_my = my * m_per
        r_left = left * m_per
        r_right = right * m_per
        r_opp = opp * m_per

        def rc(src, dst, s, r, dev):
            return pltpu.make_async_remote_copy(
                src_ref=src, dst_ref=dst,
                send_sem=ag_ss.at[s], recv_sem=ag_rs.at[r],
                device_id=(dev,), device_id_type=pl.DeviceIdType.MESH,
            )

        wcp = pltpu.make_async_copy(w_hbm, wbuf, wl_sem)
        wcp.start()

        sends = []
        for s_idx, (lo, dev) in enumerate(
            [(0, right), (HALF, right), (HALF, left), (0, left)]
        ):
            s = rc(x_hbm.at[pl.ds(lo, HALF)], xg_hbm.at[pl.ds(r_my + lo, HALF)],
                   s_idx, s_idx, dev)
            s.start()
            sends.append(s)

        def wait_recv(row, nr, rs):
            d = rc(xg_hbm.at[pl.ds(row, nr)], xg_hbm.at[pl.ds(row, nr)],
                   0, rs, my)
            d.wait_recv()

        amax_s[0] = 0.0
        wcp.wait()

        phases = [
            (r_my, HALF, None, x_hbm.at[pl.ds(0, HALF)]),
            (r_my + HALF, HALF, None, x_hbm.at[pl.ds(HALF, HALF)]),
            (r_left, HALF, 0, None),
            (r_right + HALF, HALF, 2, None),
            (r_left + HALF, HALF, 1, None),
            (r_right, HALF, 3, None),
            (r_opp, QTR, 4, None),
            (r_opp + HALF, QTR, 6, None),
            (r_opp + QTR, QTR, 5, None),
            (r_opp + HALF + QTR, QTR, 7, None),
        ]

        youts = []
        for p, (row, nr, rs, src) in enumerate(phases):
            if rs is not None:
                wait_recv(row, nr, rs)
                src = xg_hbm.at[pl.ds(row, nr)]
            if p == 2:
                for q, s_idx in ((0, 4), (1, 5)):
                    f = rc(xg_hbm.at[pl.ds(r_left + q * QTR, QTR)],
                           xg_hbm.at[pl.ds(r_left + q * QTR, QTR)],
                           s_idx, s_idx, right)
                    f.start()
                    sends.append(f)
            if p == 3:
                for q, s_idx in ((0, 6), (1, 7)):
                    f = rc(xg_hbm.at[pl.ds(r_right + HALF + q * QTR, QTR)],
                           xg_hbm.at[pl.ds(r_right + HALF + q * QTR, QTR)],
                           s_idx, s_idx, left)
                    f.start()
                    sends.append(f)

            xcp = pltpu.make_async_copy(src, xbuf.at[pl.ds(0, nr)], xl_sem)
            xcp.start()
            xcp.wait()

            if p >= 2:
                youts[p - 2].wait()
            boff = (p % 2) * HALF
            ybuf[boff:boff + nr, :] = jnp.dot(
                xbuf[0:nr, :], wbuf[...], preferred_element_type=jnp.float32,
            )
            amax_s[0] = jnp.maximum(
                amax_s[0], jnp.max(jnp.abs(ybuf[boff:boff + nr, :]))
            )
            yo = pltpu.make_async_copy(
                ybuf.at[pl.ds(boff, nr)], out_hbm.at[pl.ds(row, nr)],
                yo_sems.at[p % 2],
            )
            yo.start()
            youts.append(yo)

        youts[-2].wait()
        youts[-1].wait()

        lds = []
        for i in range(2):
            ld = pltpu.make_async_copy(
                out_hbm.at[pl.ds(i * HALF, HALF)], ybuf.at[pl.ds(i * HALF, HALF)],
                ep_ld.at[i],
            )
            ld.start()
            lds.append(ld)

        vbuf[...] = jnp.full((8, 128), amax_s[0], jnp.float32)
        acp = pltpu.make_async_copy(vbuf, gbuf.at[my], acp_sem)
        acp.start()
        acp.wait()
        asends = []
        for d in range(1, N_DEV):
            tgt = lax.rem(my + d, N_DEV)
            a = pltpu.make_async_remote_copy(
                src_ref=vbuf, dst_ref=gbuf.at[my],
                send_sem=am_ss.at[d - 1], recv_sem=am_rs.at[d - 1],
                device_id=(tgt,), device_id_type=pl.DeviceIdType.MESH,
            )
            a.start()
            asends.append(a)
        for a in asends:
            a.wait_send()
        for d in range(1, N_DEV):
            src_slot = lax.rem(my - d + N_DEV, N_DEV)
            a = pltpu.make_async_remote_copy(
                src_ref=vbuf, dst_ref=gbuf.at[src_slot],
                send_sem=am_ss.at[d - 1], recv_sem=am_rs.at[d - 1],
                device_id=(my,), device_id_type=pl.DeviceIdType.MESH,
            )
            a.wait_recv()
        scale = jnp.max(gbuf[...]) / 448.0

        nb = m // HALF
        sts = []
        for b in range(nb):
            boff = (b % 2) * HALF
            lds[b].wait()
            z = jnp.clip(ybuf[boff:boff + HALF, :] / scale, -448.0, 448.0)
            ybuf[boff:boff + HALF, :] = (
                z.astype(jnp.float8_e4m3fn).astype(jnp.float32) * scale
            )
            st = pltpu.make_async_copy(
                ybuf.at[pl.ds(boff, HALF)], out_hbm.at[pl.ds(b * HALF, HALF)],
                ep_st.at[b % 2],
            )
            st.start()
            sts.append(st)
            if b + 2 < nb:
                st.wait()
                ld = pltpu.make_async_copy(
                    out_hbm.at[pl.ds((b + 2) * HALF, HALF)],
                    ybuf.at[pl.ds(boff, HALF)], ep_ld.at[b % 2],
                )
                ld.start()
                lds.append(ld)
        sts[-2].wait()
        sts[-1].wait()

        for s in sends:
            s.wait_send()

    out, _ = pl.pallas_call(
        body,
        out_shape=[
            jax.ShapeDtypeStruct((m, n_per), jnp.float32),
            jax.ShapeDtypeStruct((m, k), jnp.float32),
        ],
        in_specs=[
            pl.BlockSpec(memory_space=pltpu.MemorySpace.HBM),
            pl.BlockSpec(memory_space=pltpu.MemorySpace.HBM),
        ],
        out_specs=[
            pl.BlockSpec(memory_space=pltpu.MemorySpace.HBM),
            pl.BlockSpec(memory_space=pltpu.MemorySpace.HBM),
        ],
        scratch_shapes=[
            pltpu.VMEM((k, n_per), jnp.float32),
            pltpu.VMEM((HALF, k), jnp.float32),
            pltpu.VMEM((2 * HALF, n_per), jnp.float32),
            pltpu.VMEM((8, 128), jnp.float32),
            pltpu.VMEM((N_DEV, 8, 128), jnp.float32),
            pltpu.SMEM((1,), jnp.float32),
            pltpu.SemaphoreType.DMA((8,)),
            pltpu.SemaphoreType.DMA((8,)),
            pltpu.SemaphoreType.DMA,
            pltpu.SemaphoreType.DMA,
            pltpu.SemaphoreType.DMA((2,)),
            pltpu.SemaphoreType.DMA,
            pltpu.SemaphoreType.DMA((3,)),
            pltpu.SemaphoreType.DMA((3,)),
            pltpu.SemaphoreType.DMA((2,)),
            pltpu.SemaphoreType.DMA((2,)),
        ],
        compiler_params=pltpu.CompilerParams(
            vmem_limit_bytes=60 * 1024 * 1024,
        ),
    )(x, w_mat)
    return out
